# Initial kernel scaffold; baseline (speedup 1.0000x reference)
import jax
import jax.numpy as jnp
from jax.experimental import pallas as pl


def kernel(user_ids, movie_ids, user_table, movie_table, W, b):
    raise NotImplementedError("write your pallas kernel here")


# speedup vs baseline: 1.3745x; 1.3745x over previous
import functools

import jax
import jax.numpy as jnp
from jax import lax
from jax.experimental import pallas as pl
from jax.experimental.pallas import tpu as pltpu
from jax.experimental.pallas import tpu_sc as plsc

NUM_USERS = 100000
NUM_MOVIES = 100000
EMBED_DIM = 64
BATCH = 16384

NC = 2
NS = 16
NW = NC * NS
BPW = BATCH // NW
CHUNK = 128
NCH = BPW // CHUNK


@functools.partial(
    pl.kernel,
    out_type=jax.ShapeDtypeStruct((BATCH,), jnp.float32),
    mesh=plsc.VectorSubcoreMesh(core_axis_name="c", subcore_axis_name="s"),
    compiler_params=pltpu.CompilerParams(
        needs_layout_passes=False, use_tc_tiling_on_sc=False),
    scratch_types=[
        pltpu.VMEM((NCH, CHUNK), jnp.int32),
        pltpu.VMEM((NCH, CHUNK), jnp.int32),
        pltpu.VMEM((BPW, EMBED_DIM), jnp.float32),
        pltpu.VMEM((BPW, EMBED_DIM), jnp.float32),
        pltpu.VMEM((2 * EMBED_DIM,), jnp.float32),
        pltpu.VMEM((16,), jnp.float32),
        pltpu.VMEM((BPW,), jnp.float32),
        pltpu.SemaphoreType.DMA,
    ],
)
def _sc_embed(uid_hbm, mid_hbm, ut_hbm, mt_hbm, w_hbm, b_hbm, out_hbm,
              uidx_v, midx_v, urows_v, mrows_v, w_v, b_v, out_v, sem):
    wid = lax.axis_index("s") * NC + lax.axis_index("c")
    base = wid * BPW

    pltpu.sync_copy(uid_hbm.at[pl.ds(wid * NCH, NCH)], uidx_v)
    pltpu.sync_copy(mid_hbm.at[pl.ds(wid * NCH, NCH)], midx_v)
    pltpu.sync_copy(w_hbm, w_v)
    pltpu.sync_copy(b_hbm, b_v)

    copies = []
    for k in range(NCH):
        copies.append(pltpu.async_copy(
            ut_hbm.at[uidx_v.at[k]], urows_v.at[pl.ds(k * CHUNK, CHUNK)], sem))
        copies.append(pltpu.async_copy(
            mt_hbm.at[midx_v.at[k]], mrows_v.at[pl.ds(k * CHUNK, CHUNK)], sem))
    for cp in copies:
        cp.wait()

    wu = [w_v[pl.ds(i * 16, 16)] for i in range(4)]
    wm = [w_v[pl.ds(EMBED_DIM + i * 16, 16)] for i in range(4)]
    bias = b_v[pl.ds(0, 16)][0]

    lane = lax.iota(jnp.int32, 16)

    def group_body(g, carry):
        res = jnp.zeros(16, jnp.float32)
        for j in range(16):
            r = g * 16 + j
            acc = urows_v[r, pl.ds(0, 16)] * wu[0] + mrows_v[r, pl.ds(0, 16)] * wm[0]
            for i in range(1, 4):
                acc = acc + urows_v[r, pl.ds(i * 16, 16)] * wu[i]
                acc = acc + mrows_v[r, pl.ds(i * 16, 16)] * wm[i]
            res = jnp.where(lane == j, jnp.sum(acc) + bias, res)
        out_v[pl.ds(g * 16, 16)] = res
        return carry

    lax.fori_loop(0, BPW // 16, group_body, 0)

    pltpu.sync_copy(out_v, out_hbm.at[pl.ds(base, BPW)])


def kernel(user_ids, movie_ids, user_table, movie_table, W, b):
    uid2 = user_ids.astype(jnp.int32).reshape(NW * NCH, CHUNK)
    mid2 = movie_ids.astype(jnp.int32).reshape(NW * NCH, CHUNK)
    w_flat = W.astype(jnp.float32).reshape(2 * EMBED_DIM)
    b16 = jnp.broadcast_to(b.astype(jnp.float32), (16,))
    return _sc_embed(uid2, mid2, user_table, movie_table, w_flat, b16)
